# baseline (speedup 1.0000x reference)
import functools

import jax
import jax.numpy as jnp
from jax import lax
from jax.experimental import pallas as pl
from jax.experimental.pallas import tpu as pltpu
from jax.experimental.pallas import tpu_sc as plsc

C = 64
NX, NY = 432, 496
NPOS = NX * NY
P = 4512
L = 16
NC, NS = 2, 16
NW = NC * NS
W = NPOS // NW
PV = P // L
PID_BITS = 13
KEY_SENT = 1 << 26
CHUNK = 128
CAPW = 256
NCHB = CAPW * C // CHUNK

_MESH = plsc.VectorSubcoreMesh(
    core_axis_name="c", subcore_axis_name="s", num_cores=NC, num_subcores=NS
)

_ZB = 6912


def _zero_body(o_ref):
    o_ref[...] = jnp.zeros_like(o_ref)


_zero_fill = pl.pallas_call(
    _zero_body,
    out_shape=jax.ShapeDtypeStruct((C, NPOS), jnp.float32),
    grid=(NPOS // _ZB,),
    out_specs=pl.BlockSpec((C, _ZB), lambda i: (0, i)),
)


@functools.partial(
    pl.kernel,
    out_type=(),
    mesh=_MESH,
    scratch_types=[
        pltpu.VMEM((P * 4,), jnp.int32),
        pltpu.VMEM((P + 2 * L,), jnp.int32),
        pltpu.VMEM((P + 2 * L,), jnp.int32),
        pltpu.VMEM((P + 2 * L,), jnp.int32),
        pltpu.VMEM((P + 2 * L,), jnp.int32),
        pltpu.VMEM((W,), jnp.int32),
        pltpu.VMEM((P + 2 * L,), jnp.int32),
        pltpu.VMEM((P + 2 * L,), jnp.int32),
        pltpu.VMEM((CAPW, 2 * C), jnp.float32),
        pltpu.VMEM((NCHB, CHUNK), jnp.int32),
        pltpu.VMEM((NCHB, CHUNK), jnp.float32),
        pltpu.VMEM((2 * L,), jnp.int32),
        pltpu.SemaphoreType.DMA,
        pltpu.SemaphoreType.DMA,
    ],
    compiler_params=pltpu.CompilerParams(needs_layout_passes=False),
)
def _scatter_kernel(pf_hbm, coords_hbm, canvas_hbm,
                    coords_v, keyw, colb, pidb, iwinb, wslab, wcolb, wpidb,
                    rowbuf, idxsc, paysc, nxtb, sem_row, sem_sc):
    w = lax.axis_index("s") * NC + lax.axis_index("c")
    lo = w * W
    iota = lax.iota(jnp.int32, L)
    zeros16i = jnp.zeros((L,), jnp.int32)

    pltpu.sync_copy(coords_hbm, coords_v)

    def scan_body(i, cnt):
        p0 = i * L
        base4 = (p0 + iota) * 4
        c1 = plsc.load_gather(coords_v, [base4 + 1])
        c2 = plsc.load_gather(coords_v, [base4 + 2])
        c3 = plsc.load_gather(coords_v, [base4 + 3])
        idx = c1 + c2 * NX + c3
        m = (idx >= lo) & (idx < lo + W)
        key = (idx - lo) * (1 << PID_BITS) + (p0 + iota)
        mi = m.astype(jnp.int32)
        pos = cnt + plsc.cumsum(mi) - 1
        pos = jnp.where(m, pos, 0)
        plsc.store_scatter(keyw, [pos], key, mask=m)
        return cnt + jnp.sum(mi)

    nmemb = lax.fori_loop(0, PV, scan_body, jnp.int32(0))
    plsc.store_scatter(keyw, [nmemb + iota],
                       jnp.full((L,), KEY_SENT, jnp.int32))
    ngrp = (nmemb + L - 1) // L
    nxtb[pl.ds(L, L)] = jnp.full((L,), KEY_SENT, jnp.int32)

    def prep(g, _):
        kv = keyw[pl.ds(g * L, L)]
        sk, _sv = plsc.sort_key_val(kv, kv)
        nxtb[pl.ds(0, L)] = sk
        nxt = nxtb[pl.ds(1, L)]
        win = ((sk >> PID_BITS) != (nxt >> PID_BITS)) & (sk < KEY_SENT)
        col = jnp.minimum(sk >> PID_BITS, W - 1)
        pid = sk & ((1 << PID_BITS) - 1)
        at = g * L + iota
        plsc.store_scatter(colb, [at], col)
        plsc.store_scatter(pidb, [at], pid)
        plsc.store_scatter(iwinb, [at], win.astype(jnp.int32))
        plsc.store_scatter(wslab, [col], pid, mask=win)
        return 0

    lax.fori_loop(0, ngrp, prep, 0)

    def wchk(g, cnt):
        at = g * L + iota
        col = plsc.load_gather(colb, [at])
        pid = plsc.load_gather(pidb, [at])
        iwin = plsc.load_gather(iwinb, [at]) != 0
        gwin = iwin & (plsc.load_gather(wslab, [col]) == pid)
        gi = gwin.astype(jnp.int32)
        pos = cnt + plsc.cumsum(gi) - 1
        pos = jnp.where(gwin, pos, 0)
        plsc.store_scatter(wcolb, [pos], col, mask=gwin)
        plsc.store_scatter(wpidb, [pos], pid, mask=gwin)
        return cnt + jnp.sum(gi)

    nwin = lax.fori_loop(0, ngrp, wchk, jnp.int32(0))

    @pl.when(nwin > 0)
    def _():
        c0 = plsc.load_gather(wcolb, [zeros16i])
        p0 = plsc.load_gather(wpidb, [zeros16i])
        plsc.store_scatter(wcolb, [nwin + iota], c0)
        plsc.store_scatter(wpidb, [nwin + iota], p0)

        nwin_pad = ((nwin + L - 1) // L) * L
        nbatch = (nwin_pad + CAPW - 1) // CAPW

        def batch_body(b, _):
            r0 = b * CAPW
            nb = jnp.minimum(nwin_pad - r0, CAPW)
            nrch = nb // L

            def gfire(ch, _):
                pidv = wpidb[pl.ds(r0 + ch * L, L)]
                pltpu.async_copy(pf_hbm.at[pidv >> 1],
                                 rowbuf.at[pl.ds(ch * L, L), :], sem_row)
                return 0
            lax.fori_loop(0, nrch, gfire, 0)

            def gdrain(ch, _):
                pltpu.make_async_copy(
                    pf_hbm.at[pl.ds(0, L), :],
                    rowbuf.at[pl.ds(ch * L, L), :], sem_row).wait()
                return 0
            lax.fori_loop(0, nrch, gdrain, 0)

            def bld(ch, _):
                rl = ch * L + iota
                colv = wcolb[pl.ds(r0 + ch * L, L)]
                pidv = wpidb[pl.ds(r0 + ch * L, L)]
                half = (pidv & 1) * C
                wordv = colv + lo
                for c in range(C):
                    t = rl * C + c
                    tch = t >> 7
                    tln = t & (CHUNK - 1)
                    plsc.store_scatter(idxsc, [tch, tln], wordv + c * NPOS)
                    pay = plsc.load_gather(rowbuf, [rl, half + c])
                    plsc.store_scatter(paysc, [tch, tln], pay)
                return 0
            lax.fori_loop(0, nrch, bld, 0)

            nch = nb * C // CHUNK

            def sfire(ch, _):
                pltpu.async_copy(paysc.at[ch],
                                 canvas_hbm.at[idxsc.at[ch]], sem_sc)
                return 0
            lax.fori_loop(0, nch, sfire, 0)

            def sdrain(ch, _):
                pltpu.make_async_copy(paysc.at[ch],
                                      canvas_hbm.at[idxsc.at[ch]],
                                      sem_sc).wait()
                return 0
            lax.fori_loop(0, nch, sdrain, 0)
            return 0

        lax.fori_loop(0, nbatch, batch_body, 0)


def kernel(pillar_features, coords):
    pf_pairs = pillar_features.reshape(P // 2, 2 * C)
    coords_flat = coords.reshape(P * 4).astype(jnp.int32)
    canvas_ref = jax.new_ref(_zero_fill().reshape(C * NPOS))
    _scatter_kernel(pf_pairs, coords_flat, canvas_ref)
    return canvas_ref[...].reshape(1, C, NY, NX)


# speedup vs baseline: 1.0689x; 1.0689x over previous
import functools

import jax
import jax.numpy as jnp
from jax import lax
from jax.experimental import pallas as pl
from jax.experimental.pallas import tpu as pltpu
from jax.experimental.pallas import tpu_sc as plsc

C = 64
NX, NY = 432, 496
NPOS = NX * NY
P = 4512
L = 16
NC, NS = 2, 16
NW = NC * NS
W = NPOS // NW
PV = P // L
PID_BITS = 13
KEY_SENT = 1 << 26
CHUNK = 128
CAPW = 256
NCHB = CAPW * C // CHUNK

_MESH = plsc.VectorSubcoreMesh(
    core_axis_name="c", subcore_axis_name="s", num_cores=NC, num_subcores=NS
)

_ZB = C * NPOS // 31


def _zero_body(o_ref):
    o_ref[...] = jnp.zeros_like(o_ref)


_zero_fill = pl.pallas_call(
    _zero_body,
    out_shape=jax.ShapeDtypeStruct((C * NPOS,), jnp.float32),
    grid=(31,),
    out_specs=pl.BlockSpec((_ZB,), lambda i: (i,)),
)


@functools.partial(
    pl.kernel,
    out_type=(),
    mesh=_MESH,
    scratch_types=[
        pltpu.VMEM((P * 4,), jnp.int32),
        pltpu.VMEM((P + 2 * L,), jnp.int32),
        pltpu.VMEM((P + 2 * L,), jnp.int32),
        pltpu.VMEM((P + 2 * L,), jnp.int32),
        pltpu.VMEM((P + 2 * L,), jnp.int32),
        pltpu.VMEM((W,), jnp.int32),
        pltpu.VMEM((P + 2 * L,), jnp.int32),
        pltpu.VMEM((P + 2 * L,), jnp.int32),
        pltpu.VMEM((CAPW, 2 * C), jnp.float32),
        pltpu.VMEM((NCHB, CHUNK), jnp.int32),
        pltpu.VMEM((NCHB, CHUNK), jnp.float32),
        pltpu.VMEM((2 * L,), jnp.int32),
        pltpu.SemaphoreType.DMA,
        pltpu.SemaphoreType.DMA,
    ],
    compiler_params=pltpu.CompilerParams(needs_layout_passes=False),
)
def _scatter_kernel(pf_hbm, coords_hbm, canvas_hbm,
                    coords_v, keyw, colb, pidb, iwinb, wslab, wcolb, wpidb,
                    rowbuf, idxsc, paysc, nxtb, sem_row, sem_sc):
    w = lax.axis_index("s") * NC + lax.axis_index("c")
    lo = w * W
    iota = lax.iota(jnp.int32, L)
    zeros16i = jnp.zeros((L,), jnp.int32)

    with jax.named_scope("ph_coords"):
        pltpu.sync_copy(coords_hbm, coords_v)

    def scan_body(i, cnt):
        p0 = i * L
        base4 = (p0 + iota) * 4
        c1 = plsc.load_gather(coords_v, [base4 + 1])
        c2 = plsc.load_gather(coords_v, [base4 + 2])
        c3 = plsc.load_gather(coords_v, [base4 + 3])
        idx = c1 + c2 * NX + c3
        m = (idx >= lo) & (idx < lo + W)
        key = (idx - lo) * (1 << PID_BITS) + (p0 + iota)
        mi = m.astype(jnp.int32)
        pos = cnt + plsc.cumsum(mi) - 1
        pos = jnp.where(m, pos, 0)
        plsc.store_scatter(keyw, [pos], key, mask=m)
        return cnt + jnp.sum(mi)

    with jax.named_scope("ph_scan"):
        nmemb = lax.fori_loop(0, PV, scan_body, jnp.int32(0))
    plsc.store_scatter(keyw, [nmemb + iota],
                       jnp.full((L,), KEY_SENT, jnp.int32))
    ngrp = (nmemb + L - 1) // L
    nxtb[pl.ds(L, L)] = jnp.full((L,), KEY_SENT, jnp.int32)

    def prep(g, _):
        kv = keyw[pl.ds(g * L, L)]
        sk, _sv = plsc.sort_key_val(kv, kv)
        nxtb[pl.ds(0, L)] = sk
        nxt = nxtb[pl.ds(1, L)]
        win = ((sk >> PID_BITS) != (nxt >> PID_BITS)) & (sk < KEY_SENT)
        col = jnp.minimum(sk >> PID_BITS, W - 1)
        pid = sk & ((1 << PID_BITS) - 1)
        at = g * L + iota
        plsc.store_scatter(colb, [at], col)
        plsc.store_scatter(pidb, [at], pid)
        plsc.store_scatter(iwinb, [at], win.astype(jnp.int32))
        plsc.store_scatter(wslab, [col], pid, mask=win)
        return 0

    with jax.named_scope("ph_prep"):
        lax.fori_loop(0, ngrp, prep, 0)

    def wchk(g, cnt):
        at = g * L + iota
        col = plsc.load_gather(colb, [at])
        pid = plsc.load_gather(pidb, [at])
        iwin = plsc.load_gather(iwinb, [at]) != 0
        gwin = iwin & (plsc.load_gather(wslab, [col]) == pid)
        gi = gwin.astype(jnp.int32)
        pos = cnt + plsc.cumsum(gi) - 1
        pos = jnp.where(gwin, pos, 0)
        plsc.store_scatter(wcolb, [pos], col, mask=gwin)
        plsc.store_scatter(wpidb, [pos], pid, mask=gwin)
        return cnt + jnp.sum(gi)

    with jax.named_scope("ph_wchk"):
        nwin = lax.fori_loop(0, ngrp, wchk, jnp.int32(0))

    @pl.when(nwin > 0)
    def _():
        c0 = plsc.load_gather(wcolb, [zeros16i])
        p0 = plsc.load_gather(wpidb, [zeros16i])
        plsc.store_scatter(wcolb, [nwin + iota], c0)
        plsc.store_scatter(wpidb, [nwin + iota], p0)

        nwin_pad = ((nwin + L - 1) // L) * L
        nbatch = (nwin_pad + CAPW - 1) // CAPW

        def batch_body(b, _):
            r0 = b * CAPW
            nb = jnp.minimum(nwin_pad - r0, CAPW)
            nrch = nb // L

            def gfire(ch, _):
                pidv = wpidb[pl.ds(r0 + ch * L, L)]
                pltpu.async_copy(pf_hbm.at[pidv >> 1],
                                 rowbuf.at[pl.ds(ch * L, L), :], sem_row)
                return 0
            with jax.named_scope("ph_gfire"):
                lax.fori_loop(0, nrch, gfire, 0)

            def gdrain(ch, _):
                pltpu.make_async_copy(
                    pf_hbm.at[pl.ds(0, L), :],
                    rowbuf.at[pl.ds(ch * L, L), :], sem_row).wait()
                return 0
            with jax.named_scope("ph_gdrain"):
                lax.fori_loop(0, nrch, gdrain, 0)

            def bld(ch, _):
                rl = ch * L + iota
                colv = wcolb[pl.ds(r0 + ch * L, L)]
                pidv = wpidb[pl.ds(r0 + ch * L, L)]
                half = (pidv & 1) * C
                wordv = colv + lo
                for c in range(C):
                    t = rl * C + c
                    tch = t >> 7
                    tln = t & (CHUNK - 1)
                    plsc.store_scatter(idxsc, [tch, tln], wordv + c * NPOS)
                    pay = plsc.load_gather(rowbuf, [rl, half + c])
                    plsc.store_scatter(paysc, [tch, tln], pay)
                return 0
            with jax.named_scope("ph_bld"):
                lax.fori_loop(0, nrch, bld, 0)

            nch = nb * C // CHUNK

            def sfire(ch, _):
                pltpu.async_copy(paysc.at[ch],
                                 canvas_hbm.at[idxsc.at[ch]], sem_sc)
                return 0
            with jax.named_scope("ph_sfire"):
                lax.fori_loop(0, nch, sfire, 0)

            def sdrain(ch, _):
                pltpu.make_async_copy(paysc.at[ch],
                                      canvas_hbm.at[idxsc.at[ch]],
                                      sem_sc).wait()
                return 0
            with jax.named_scope("ph_sdrain"):
                lax.fori_loop(0, nch, sdrain, 0)
            return 0

        lax.fori_loop(0, nbatch, batch_body, 0)


def kernel(pillar_features, coords):
    pf_pairs = pillar_features.reshape(P // 2, 2 * C)
    coords_flat = coords.reshape(P * 4).astype(jnp.int32)
    canvas_ref = jax.new_ref(_zero_fill())
    _scatter_kernel(pf_pairs, coords_flat, canvas_ref)
    return canvas_ref[...].reshape(1, C, NY, NX)
